# baseline (speedup 1.0000x reference)
import jax
import jax.numpy as jnp
from jax import lax
from jax.experimental import pallas as pl
from jax.experimental.pallas import tpu as pltpu
from jax.experimental.pallas import tpu_sc as plsc

N_NODES = 10000
N_EDGES = 320000
FEAT = 128

NC = 2
NS = 16
NW = NC * NS

CHUNK = 128
CHUNKS_PER_W = 80
N_PHASES = 2
CHUNKS_PER_PHASE = CHUNKS_PER_W // N_PHASES
EDGES_PER_W = CHUNK * CHUNKS_PER_W
E_PAD = NW * EDGES_PER_W
DUMMY_ROW = N_NODES

ACC_ROWS = 10240
ROWS_PER_TILE = ACC_ROWS // NS


def _sc_body(x_hbm, srcp_hbm, dstp_hbm, part_hbm,
             src_v, dst_v, bufs, acc_sh, gsem, ssem):
    cid = lax.axis_index("c")
    sid = lax.axis_index("s")
    wid = sid * NC + cid

    zv = jnp.zeros((16,), jnp.float32)

    def zrow(i, c):
        for jj in range(FEAT // 16):
            bufs[0, i, pl.ds(jj * 16, 16)] = zv
        return c

    lax.fori_loop(0, CHUNK, zrow, 0)
    for j in range(ROWS_PER_TILE // CHUNK):
        pltpu.sync_copy(bufs.at[0],
                        acc_sh.at[pl.ds(sid * ROWS_PER_TILE + j * CHUNK, CHUNK)])
    plsc.subcore_barrier()

    for ph in range(N_PHASES):
        pltpu.sync_copy(srcp_hbm.at[wid, ph], src_v)
        pltpu.sync_copy(dstp_hbm.at[wid, ph], dst_v)

        def step(t, c):
            pltpu.async_copy(x_hbm.at[src_v.at[t]], bufs.at[0], gsem).wait()
            pltpu.sync_copy(bufs.at[0], acc_sh.at[dst_v.at[t]], add=True)
            return c

        lax.fori_loop(0, CHUNKS_PER_PHASE, step, 0)
    plsc.subcore_barrier()

    pltpu.sync_copy(acc_sh.at[pl.ds(sid * ROWS_PER_TILE, ROWS_PER_TILE)],
                    part_hbm.at[cid, pl.ds(sid * ROWS_PER_TILE, ROWS_PER_TILE)])


def _scatter_partials(x, srcp, dstp):
    mesh = plsc.VectorSubcoreMesh(core_axis_name="c", subcore_axis_name="s",
                                  num_cores=NC, num_subcores=NS)
    return pl.kernel(
        _sc_body,
        out_type=jax.ShapeDtypeStruct((NC, ACC_ROWS, FEAT), jnp.float32),
        mesh=mesh,
        scratch_types=[
            pltpu.VMEM((CHUNKS_PER_PHASE, CHUNK), jnp.int32),
            pltpu.VMEM((CHUNKS_PER_PHASE, CHUNK), jnp.int32),
            pltpu.VMEM((1, CHUNK, FEAT), jnp.float32),
            pltpu.VMEM_SHARED((ACC_ROWS, FEAT), jnp.float32),
            pltpu.SemaphoreType.DMA,
            pltpu.SemaphoreType.DMA,
        ],
    )(x, srcp, dstp)


def _tc_body(p_ref, x_ref, w_ref, l_ref, b_ref, o_ref):
    a = p_ref[0] + p_ref[1]
    o_ref[...] = (
        jnp.dot(a, w_ref[...], preferred_element_type=jnp.float32)
        + jnp.dot(x_ref[...], l_ref[...], preferred_element_type=jnp.float32)
        + b_ref[...]
    )


def _combine(partials, x, weight, loop_weight, h_bias):
    blk = 1000
    grid = N_NODES // blk
    return pl.pallas_call(
        _tc_body,
        grid=(grid,),
        in_specs=[
            pl.BlockSpec((NC, blk, FEAT), lambda i: (0, i, 0)),
            pl.BlockSpec((blk, FEAT), lambda i: (i, 0)),
            pl.BlockSpec((FEAT, FEAT), lambda i: (0, 0)),
            pl.BlockSpec((FEAT, FEAT), lambda i: (0, 0)),
            pl.BlockSpec((1, FEAT), lambda i: (0, 0)),
        ],
        out_specs=pl.BlockSpec((blk, FEAT), lambda i: (i, 0)),
        out_shape=jax.ShapeDtypeStruct((N_NODES, FEAT), jnp.float32),
    )(partials, x, weight, loop_weight, h_bias.reshape(1, FEAT))


def kernel(x, edge_index, weight, h_bias, loop_weight):
    src = edge_index[0].astype(jnp.int32)
    dst = edge_index[1].astype(jnp.int32)
    pad = E_PAD - N_EDGES
    srcp = jnp.concatenate([src, jnp.zeros((pad,), jnp.int32)]).reshape(
        NW, N_PHASES, CHUNKS_PER_PHASE, CHUNK)
    dstp = jnp.concatenate([dst, jnp.full((pad,), DUMMY_ROW, jnp.int32)]).reshape(
        NW, N_PHASES, CHUNKS_PER_PHASE, CHUNK)
    partials = _scatter_partials(x, srcp, dstp)
    return _combine(partials, x, weight, loop_weight, h_bias)


# speedup vs baseline: 2.9313x; 2.9313x over previous
import jax
import jax.numpy as jnp
from jax import lax
from jax.experimental import pallas as pl
from jax.experimental.pallas import tpu as pltpu
from jax.experimental.pallas import tpu_sc as plsc

N_NODES = 10000
N_EDGES = 320000
FEAT = 128

NC = 2
NS = 16
NW = NC * NS

CHUNK = 128
CHUNKS_PER_W = 80
N_PHASES = 2
CHUNKS_PER_PHASE = CHUNKS_PER_W // N_PHASES
EDGES_PER_W = CHUNK * CHUNKS_PER_W
E_PAD = NW * EDGES_PER_W
DUMMY_ROW = N_NODES

ACC_ROWS = 10240
ROWS_PER_TILE = ACC_ROWS // NS


def _sc_body(x_hbm, srcp_hbm, dstp_hbm, part_hbm,
             src_v, dst_v, bufs, acc_sh, gsem, ssem):
    cid = lax.axis_index("c")
    sid = lax.axis_index("s")
    wid = sid * NC + cid

    zv = jnp.zeros((16,), jnp.float32)

    def zrow(i, c):
        for jj in range(FEAT // 16):
            bufs[0, i, pl.ds(jj * 16, 16)] = zv
        return c

    lax.fori_loop(0, CHUNK, zrow, 0)
    for j in range(ROWS_PER_TILE // CHUNK):
        pltpu.sync_copy(bufs.at[0],
                        acc_sh.at[pl.ds(sid * ROWS_PER_TILE + j * CHUNK, CHUNK)])
    plsc.subcore_barrier()

    for ph in range(N_PHASES):
        pltpu.sync_copy(srcp_hbm.at[wid, ph], src_v)
        pltpu.sync_copy(dstp_hbm.at[wid, ph], dst_v)

        def step(t, c):
            pltpu.async_copy(x_hbm.at[src_v.at[t]], bufs.at[0], gsem).wait()
            pltpu.sync_copy(bufs.at[0], acc_sh.at[dst_v.at[t]], add=True)
            return c

        lax.fori_loop(0, CHUNKS_PER_PHASE, step, 0)
    plsc.subcore_barrier()

    pltpu.sync_copy(acc_sh.at[pl.ds(sid * ROWS_PER_TILE, ROWS_PER_TILE)],
                    part_hbm.at[cid, pl.ds(sid * ROWS_PER_TILE, ROWS_PER_TILE)])


def _scatter_partials(x, srcp, dstp):
    mesh = plsc.VectorSubcoreMesh(core_axis_name="c", subcore_axis_name="s",
                                  num_cores=NC, num_subcores=NS)
    return pl.kernel(
        _sc_body,
        out_type=jax.ShapeDtypeStruct((NC, ACC_ROWS, FEAT), jnp.float32),
        mesh=mesh,
        scratch_types=[
            pltpu.VMEM((CHUNKS_PER_PHASE, CHUNK), jnp.int32),
            pltpu.VMEM((CHUNKS_PER_PHASE, CHUNK), jnp.int32),
            pltpu.VMEM((1, CHUNK, FEAT), jnp.float32),
            pltpu.VMEM_SHARED((ACC_ROWS, FEAT), jnp.float32),
            pltpu.SemaphoreType.DMA,
            pltpu.SemaphoreType.DMA,
        ],
    )(x, srcp, dstp)


def _tc_body(p_ref, x_ref, w_ref, l_ref, b_ref, o_ref):
    a = p_ref[0] + p_ref[1]
    o_ref[...] = (
        jnp.dot(a, w_ref[...], preferred_element_type=jnp.float32)
        + jnp.dot(x_ref[...], l_ref[...], preferred_element_type=jnp.float32)
        + b_ref[...]
    )


def _combine(partials, x, weight, loop_weight, h_bias):
    blk = 1000
    grid = N_NODES // blk
    return pl.pallas_call(
        _tc_body,
        grid=(grid,),
        in_specs=[
            pl.BlockSpec((NC, blk, FEAT), lambda i: (0, i, 0)),
            pl.BlockSpec((blk, FEAT), lambda i: (i, 0)),
            pl.BlockSpec((FEAT, FEAT), lambda i: (0, 0)),
            pl.BlockSpec((FEAT, FEAT), lambda i: (0, 0)),
            pl.BlockSpec((1, FEAT), lambda i: (0, 0)),
        ],
        out_specs=pl.BlockSpec((blk, FEAT), lambda i: (i, 0)),
        out_shape=jax.ShapeDtypeStruct((N_NODES, FEAT), jnp.float32),
    )(partials, x, weight, loop_weight, h_bias.reshape(1, FEAT))


def kernel(x, edge_index, weight, h_bias, loop_weight):
    src = edge_index[0].astype(jnp.int32)
    dst = edge_index[1].astype(jnp.int32)
    pad = E_PAD - N_EDGES
    ar = jnp.arange(pad, dtype=jnp.int32)
    pad_src = ar % N_NODES
    pad_dst = DUMMY_ROW + 1 + ar % (ACC_ROWS - N_NODES - 1)
    srcp = jnp.concatenate([src, pad_src]).reshape(
        NW, N_PHASES, CHUNKS_PER_PHASE, CHUNK)
    dstp = jnp.concatenate([dst, pad_dst]).reshape(
        NW, N_PHASES, CHUNKS_PER_PHASE, CHUNK)
    partials = _scatter_partials(x, srcp, dstp)
    return _combine(partials, x, weight, loop_weight, h_bias)


# speedup vs baseline: 3.3294x; 1.1358x over previous
import jax
import jax.numpy as jnp
from jax import lax
from jax.experimental import pallas as pl
from jax.experimental.pallas import tpu as pltpu
from jax.experimental.pallas import tpu_sc as plsc

N_NODES = 10000
N_EDGES = 320000
FEAT = 128

NC = 2
NS = 16
NW = NC * NS

CHUNK = 128
CHUNKS_PER_W = 80
N_PHASES = 2
CHUNKS_PER_PHASE = CHUNKS_PER_W // N_PHASES
EDGES_PER_W = CHUNK * CHUNKS_PER_W
E_PAD = NW * EDGES_PER_W
DUMMY_ROW = N_NODES

ACC_ROWS = 10240
ROWS_PER_TILE = ACC_ROWS // NS


def _sc_body(x_hbm, srcp_hbm, dstp_hbm, part_hbm,
             src_v, dst_v, bufs, acc_sh, gsem, ssem):
    cid = lax.axis_index("c")
    sid = lax.axis_index("s")
    wid = sid * NC + cid

    zv = jnp.zeros((16,), jnp.float32)

    def zrow(i, c):
        for jj in range(FEAT // 16):
            bufs[0, i, pl.ds(jj * 16, 16)] = zv
        return c

    lax.fori_loop(0, CHUNK, zrow, 0)
    for j in range(ROWS_PER_TILE // CHUNK):
        pltpu.sync_copy(bufs.at[0],
                        acc_sh.at[pl.ds(sid * ROWS_PER_TILE + j * CHUNK, CHUNK)])
    plsc.subcore_barrier()

    for ph in range(N_PHASES):
        pltpu.sync_copy(srcp_hbm.at[wid, ph], src_v)
        pltpu.sync_copy(dstp_hbm.at[wid, ph], dst_v)

        def step(t, c):
            j0 = 2 * t
            d0 = pltpu.async_copy(x_hbm.at[src_v.at[j0]], bufs.at[0], gsem)
            d1 = pltpu.async_copy(x_hbm.at[src_v.at[j0 + 1]], bufs.at[1], ssem)
            d0.wait()
            pltpu.sync_copy(bufs.at[0], acc_sh.at[dst_v.at[j0]], add=True)
            d1.wait()
            pltpu.sync_copy(bufs.at[1], acc_sh.at[dst_v.at[j0 + 1]], add=True)
            return c

        lax.fori_loop(0, CHUNKS_PER_PHASE // 2, step, 0)
    plsc.subcore_barrier()

    pltpu.sync_copy(acc_sh.at[pl.ds(sid * ROWS_PER_TILE, ROWS_PER_TILE)],
                    part_hbm.at[cid, pl.ds(sid * ROWS_PER_TILE, ROWS_PER_TILE)])


def _scatter_partials(x, srcp, dstp):
    mesh = plsc.VectorSubcoreMesh(core_axis_name="c", subcore_axis_name="s",
                                  num_cores=NC, num_subcores=NS)
    return pl.kernel(
        _sc_body,
        out_type=jax.ShapeDtypeStruct((NC, ACC_ROWS, FEAT), jnp.float32),
        mesh=mesh,
        scratch_types=[
            pltpu.VMEM((CHUNKS_PER_PHASE, CHUNK), jnp.int32),
            pltpu.VMEM((CHUNKS_PER_PHASE, CHUNK), jnp.int32),
            pltpu.VMEM((2, CHUNK, FEAT), jnp.float32),
            pltpu.VMEM_SHARED((ACC_ROWS, FEAT), jnp.float32),
            pltpu.SemaphoreType.DMA,
            pltpu.SemaphoreType.DMA,
        ],
    )(x, srcp, dstp)


def _tc_body(p_ref, x_ref, w_ref, l_ref, b_ref, o_ref):
    a = p_ref[0] + p_ref[1]
    o_ref[...] = (
        jnp.dot(a, w_ref[...], preferred_element_type=jnp.float32)
        + jnp.dot(x_ref[...], l_ref[...], preferred_element_type=jnp.float32)
        + b_ref[...]
    )


def _combine(partials, x, weight, loop_weight, h_bias):
    blk = 1000
    grid = N_NODES // blk
    return pl.pallas_call(
        _tc_body,
        grid=(grid,),
        in_specs=[
            pl.BlockSpec((NC, blk, FEAT), lambda i: (0, i, 0)),
            pl.BlockSpec((blk, FEAT), lambda i: (i, 0)),
            pl.BlockSpec((FEAT, FEAT), lambda i: (0, 0)),
            pl.BlockSpec((FEAT, FEAT), lambda i: (0, 0)),
            pl.BlockSpec((1, FEAT), lambda i: (0, 0)),
        ],
        out_specs=pl.BlockSpec((blk, FEAT), lambda i: (i, 0)),
        out_shape=jax.ShapeDtypeStruct((N_NODES, FEAT), jnp.float32),
    )(partials, x, weight, loop_weight, h_bias.reshape(1, FEAT))


def kernel(x, edge_index, weight, h_bias, loop_weight):
    src = edge_index[0].astype(jnp.int32)
    dst = edge_index[1].astype(jnp.int32)
    pad = E_PAD - N_EDGES
    ar = jnp.arange(pad, dtype=jnp.int32)
    pad_src = ar % N_NODES
    pad_dst = DUMMY_ROW + 1 + ar % (ACC_ROWS - N_NODES - 1)
    srcp = jnp.concatenate([src, pad_src]).reshape(
        NW, N_PHASES, CHUNKS_PER_PHASE, CHUNK)
    dstp = jnp.concatenate([dst, pad_dst]).reshape(
        NW, N_PHASES, CHUNKS_PER_PHASE, CHUNK)
    partials = _scatter_partials(x, srcp, dstp)
    return _combine(partials, x, weight, loop_weight, h_bias)


# speedup vs baseline: 3.3837x; 1.0163x over previous
import jax
import jax.numpy as jnp
from jax import lax
from jax.experimental import pallas as pl
from jax.experimental.pallas import tpu as pltpu
from jax.experimental.pallas import tpu_sc as plsc

N_NODES = 10000
N_EDGES = 320000
FEAT = 128

NC = 2
NS = 16
NW = NC * NS

CHUNK = 128
CHUNKS_PER_W = 80
N_PHASES = 2
CHUNKS_PER_PHASE = CHUNKS_PER_W // N_PHASES
EDGES_PER_W = CHUNK * CHUNKS_PER_W
E_PAD = NW * EDGES_PER_W
DUMMY_ROW = N_NODES

ACC_ROWS = 10240
ROWS_PER_TILE = ACC_ROWS // NS


def _sc_body(x_hbm, srcp_hbm, dstp_hbm, part_hbm,
             src_v, dst_v, bufs, acc_sh, gsem, ssem):
    cid = lax.axis_index("c")
    sid = lax.axis_index("s")
    wid = sid * NC + cid

    zv = jnp.zeros((16,), jnp.float32)

    def zrow(i, c):
        for jj in range(FEAT // 16):
            bufs[0, i, pl.ds(jj * 16, 16)] = zv
        return c

    lax.fori_loop(0, CHUNK, zrow, 0)
    for j in range(ROWS_PER_TILE // CHUNK):
        pltpu.sync_copy(bufs.at[0],
                        acc_sh.at[pl.ds(sid * ROWS_PER_TILE + j * CHUNK, CHUNK)])
    plsc.subcore_barrier()

    for ph in range(N_PHASES):
        pltpu.sync_copy(srcp_hbm.at[wid, ph], src_v)
        pltpu.sync_copy(dstp_hbm.at[wid, ph], dst_v)

        def step(t, c):
            j0 = 2 * t
            d0 = pltpu.async_copy(x_hbm.at[src_v.at[j0]], bufs.at[0], gsem)
            d1 = pltpu.async_copy(x_hbm.at[src_v.at[j0 + 1]], bufs.at[1], ssem)
            d0.wait()
            pltpu.sync_copy(bufs.at[0], acc_sh.at[dst_v.at[j0]], add=True)
            d1.wait()
            pltpu.sync_copy(bufs.at[1], acc_sh.at[dst_v.at[j0 + 1]], add=True)
            return c

        lax.fori_loop(0, CHUNKS_PER_PHASE // 2, step, 0)
    plsc.subcore_barrier()

    pltpu.sync_copy(acc_sh.at[pl.ds(sid * ROWS_PER_TILE, ROWS_PER_TILE)],
                    part_hbm.at[cid, pl.ds(sid * ROWS_PER_TILE, ROWS_PER_TILE)])


def _scatter_partials(x, srcp, dstp):
    mesh = plsc.VectorSubcoreMesh(core_axis_name="c", subcore_axis_name="s",
                                  num_cores=NC, num_subcores=NS)
    return pl.kernel(
        _sc_body,
        out_type=jax.ShapeDtypeStruct((NC, ACC_ROWS, FEAT), jnp.float32),
        mesh=mesh,
        scratch_types=[
            pltpu.VMEM((CHUNKS_PER_PHASE, CHUNK), jnp.int32),
            pltpu.VMEM((CHUNKS_PER_PHASE, CHUNK), jnp.int32),
            pltpu.VMEM((2, CHUNK, FEAT), jnp.float32),
            pltpu.VMEM_SHARED((ACC_ROWS, FEAT), jnp.float32),
            pltpu.SemaphoreType.DMA,
            pltpu.SemaphoreType.DMA,
        ],
    )(x, srcp, dstp)


def _tc_body(p_ref, x_ref, w_ref, l_ref, b_ref, o_ref):
    a = p_ref[0] + p_ref[1]
    o_ref[...] = (
        jnp.dot(a, w_ref[...], preferred_element_type=jnp.float32)
        + jnp.dot(x_ref[...], l_ref[...], preferred_element_type=jnp.float32)
        + b_ref[...]
    )


def _combine(partials, x, weight, loop_weight, h_bias):
    blk = 2000
    grid = N_NODES // blk
    return pl.pallas_call(
        _tc_body,
        grid=(grid,),
        in_specs=[
            pl.BlockSpec((NC, blk, FEAT), lambda i: (0, i, 0)),
            pl.BlockSpec((blk, FEAT), lambda i: (i, 0)),
            pl.BlockSpec((FEAT, FEAT), lambda i: (0, 0)),
            pl.BlockSpec((FEAT, FEAT), lambda i: (0, 0)),
            pl.BlockSpec((1, FEAT), lambda i: (0, 0)),
        ],
        out_specs=pl.BlockSpec((blk, FEAT), lambda i: (i, 0)),
        out_shape=jax.ShapeDtypeStruct((N_NODES, FEAT), jnp.float32),
    )(partials, x, weight, loop_weight, h_bias.reshape(1, FEAT))


def kernel(x, edge_index, weight, h_bias, loop_weight):
    src = edge_index[0].astype(jnp.int32)
    dst = edge_index[1].astype(jnp.int32)
    pad = E_PAD - N_EDGES
    ar = jnp.arange(pad, dtype=jnp.int32)
    pad_src = ar % N_NODES
    pad_dst = DUMMY_ROW + 1 + ar % (ACC_ROWS - N_NODES - 1)
    srcp = jnp.concatenate([src, pad_src]).reshape(
        NW, N_PHASES, CHUNKS_PER_PHASE, CHUNK)
    dstp = jnp.concatenate([dst, pad_dst]).reshape(
        NW, N_PHASES, CHUNKS_PER_PHASE, CHUNK)
    partials = _scatter_partials(x, srcp, dstp)
    return _combine(partials, x, weight, loop_weight, h_bias)
